# baseline (speedup 1.0000x reference)
import functools

import jax
import jax.numpy as jnp
from jax import lax
from jax.experimental import pallas as pl
from jax.experimental.pallas import tpu as pltpu
from jax.experimental.pallas import tpu_sc as plsc

N = 10000
E = 320000
F = 128
C = 32
G = 128

TILES = 32
K = 128
PER = E // TILES
CH = 80
PERP = CH * K
SCH = CH // 2
NPAD = 10240
RPT = NPAD // 16
ZR = 128
LANES = 16


def _sc_agg_body(xwd_hbm, rows_hbm, ews_hbm, cols_hbm, out_hbm,
                 cidx_all, ri_a, ri_b, ew_a, ew_b, buf_a, buf_b, acc_sh,
                 sem_ga, sem_gb, sem_sa, sem_sb, sem_ia, sem_ib):
    c = lax.axis_index("c")
    s = lax.axis_index("s")
    base = s * RPT
    w = c * 16 + s

    def zrow(i, _):
        for j in range(8):
            buf_a[i, pl.ds(j * LANES, LANES)] = jnp.zeros((LANES,), jnp.float32)
        return 0
    lax.fori_loop(0, ZR, zrow, 0)
    pltpu.sync_copy(cols_hbm.at[w], cidx_all)
    for i in range(5):
        pltpu.sync_copy(buf_a, acc_sh.at[pl.ds(base + i * ZR, ZR)])
    plsc.subcore_barrier()

    def fire_idx(ci, ri, ew, sem):
        pltpu.async_copy(rows_hbm.at[w, ci], ri, sem)
        pltpu.async_copy(ews_hbm.at[w, ci], ew, sem)

    def drain_idx(ci, ri, ew, sem):
        pltpu.make_async_copy(rows_hbm.at[w, ci], ri, sem).wait()
        pltpu.make_async_copy(ews_hbm.at[w, ci], ew, sem).wait()

    def fire_gather(ri, buf, sem):
        pltpu.async_copy(xwd_hbm.at[ri], buf, sem)

    def drain_gather(ri, buf, sem):
        pltpu.make_async_copy(xwd_hbm.at[ri], buf, sem).wait()

    def fire_scatter(ci, buf, sem):
        pltpu.async_copy(buf, acc_sh.at[cidx_all.at[ci]], sem, add=True)

    def drain_scatter(ci, buf, sem):
        pltpu.make_async_copy(buf, acc_sh.at[cidx_all.at[ci]], sem).wait()

    def scale(buf, ew):
        def grp(g, _):
            off = g * LANES
            ev = ew[pl.ds(off, LANES)]
            for j in range(LANES):
                e = ev[j]
                for jc in range(8):
                    sl = pl.ds(jc * LANES, LANES)
                    buf[off + j, sl] = buf[off + j, sl] * e
            return 0
        lax.fori_loop(0, K // LANES, grp, 0)

    fire_idx(0, ri_a, ew_a, sem_ia)
    fire_idx(1, ri_b, ew_b, sem_ib)
    drain_idx(0, ri_a, ew_a, sem_ia)
    fire_gather(ri_a, buf_a, sem_ga)

    def body(i, _):
        c0 = 2 * i
        c1 = 2 * i + 1
        drain_gather(ri_a, buf_a, sem_ga)
        scale(buf_a, ew_a)

        @pl.when(i > 0)
        def _():
            drain_scatter(c1, buf_b, sem_sb)
        fire_scatter(c0, buf_a, sem_sa)
        drain_idx(c1, ri_b, ew_b, sem_ib)
        fire_gather(ri_b, buf_b, sem_gb)

        @pl.when(i < SCH - 1)
        def _():
            fire_idx(c0 + 2, ri_a, ew_a, sem_ia)
        drain_gather(ri_b, buf_b, sem_gb)
        scale(buf_b, ew_b)
        drain_scatter(c0, buf_a, sem_sa)
        fire_scatter(c1, buf_b, sem_sb)

        @pl.when(i < SCH - 1)
        def _():
            drain_idx(c0 + 2, ri_a, ew_a, sem_ia)
            fire_gather(ri_a, buf_a, sem_ga)
            fire_idx(c1 + 2, ri_b, ew_b, sem_ib)
        return 0
    lax.fori_loop(0, SCH, body, 0)
    drain_scatter(CH - 1, buf_b, sem_sb)

    plsc.subcore_barrier()
    pltpu.sync_copy(acc_sh.at[pl.ds(base, RPT)],
                    out_hbm.at[c, pl.ds(base, RPT)])


def _sc_agg(xwd, rows3, ews3, cols3):
    mesh = plsc.VectorSubcoreMesh(core_axis_name="c", subcore_axis_name="s")
    fn = functools.partial(
        pl.kernel, mesh=mesh,
        out_type=jax.ShapeDtypeStruct((2, NPAD, F), jnp.float32),
        scratch_types=[
            pltpu.VMEM((CH, K), jnp.int32),
            pltpu.VMEM((K,), jnp.int32),
            pltpu.VMEM((K,), jnp.int32),
            pltpu.VMEM((K,), jnp.float32),
            pltpu.VMEM((K,), jnp.float32),
            pltpu.VMEM((K, F), jnp.float32),
            pltpu.VMEM((K, F), jnp.float32),
            pltpu.VMEM_SHARED((NPAD, F), jnp.float32),
            pltpu.SemaphoreType.DMA,
            pltpu.SemaphoreType.DMA,
            pltpu.SemaphoreType.DMA,
            pltpu.SemaphoreType.DMA,
            pltpu.SemaphoreType.DMA,
            pltpu.SemaphoreType.DMA,
        ],
    )(_sc_agg_body)
    return fn(xwd, rows3, ews3, cols3)


FD = F


def _sc_deg_body(ews_hbm, cols_hbm, out_hbm,
                 cidx_all, ew_a, ew_b, buf_a, buf_b, acc_sh,
                 sem_sa, sem_sb, sem_ia, sem_ib):
    c = lax.axis_index("c")
    s = lax.axis_index("s")
    base = s * RPT
    w = c * 16 + s

    def zrow(i, _):
        for j in range(FD // LANES):
            buf_a[i, pl.ds(j * LANES, LANES)] = jnp.zeros((LANES,), jnp.float32)
        return 0
    lax.fori_loop(0, ZR, zrow, 0)
    pltpu.sync_copy(cols_hbm.at[w], cidx_all)
    for i in range(5):
        pltpu.sync_copy(buf_a, acc_sh.at[pl.ds(base + i * ZR, ZR)])
    plsc.subcore_barrier()

    def fire_idx(ci, ew, sem):
        pltpu.async_copy(ews_hbm.at[w, ci], ew, sem)

    def drain_idx(ci, ew, sem):
        pltpu.make_async_copy(ews_hbm.at[w, ci], ew, sem).wait()

    def fire_scatter(ci, buf, sem):
        pltpu.async_copy(buf, acc_sh.at[cidx_all.at[ci]], sem, add=True)

    def drain_scatter(ci, buf, sem):
        pltpu.make_async_copy(buf, acc_sh.at[cidx_all.at[ci]], sem).wait()

    def build(buf, ew):
        def grp(g, _):
            off = g * LANES
            ev = ew[pl.ds(off, LANES)]
            for j in range(LANES):
                e16 = jnp.full((LANES,), ev[j], jnp.float32)
                for jc in range(FD // LANES):
                    buf[off + j, pl.ds(jc * LANES, LANES)] = e16
            return 0
        lax.fori_loop(0, K // LANES, grp, 0)

    fire_idx(0, ew_a, sem_ia)
    fire_idx(1, ew_b, sem_ib)

    def body(i, _):
        c0 = 2 * i
        c1 = 2 * i + 1
        drain_idx(c0, ew_a, sem_ia)
        build(buf_a, ew_a)

        @pl.when(i > 0)
        def _():
            drain_scatter(c1, buf_b, sem_sb)
        fire_scatter(c0, buf_a, sem_sa)
        drain_idx(c1, ew_b, sem_ib)
        build(buf_b, ew_b)
        drain_scatter(c0, buf_a, sem_sa)
        fire_scatter(c1, buf_b, sem_sb)

        @pl.when(i < SCH - 1)
        def _():
            fire_idx(c0 + 2, ew_a, sem_ia)
            fire_idx(c1 + 2, ew_b, sem_ib)
        return 0
    lax.fori_loop(0, SCH, body, 0)
    drain_scatter(CH - 1, buf_b, sem_sb)

    plsc.subcore_barrier()
    pltpu.sync_copy(acc_sh.at[pl.ds(base, RPT)],
                    out_hbm.at[c, pl.ds(base, RPT)])


def _sc_deg(ews3, cols3):
    mesh = plsc.VectorSubcoreMesh(core_axis_name="c", subcore_axis_name="s")
    fn = functools.partial(
        pl.kernel, mesh=mesh,
        out_type=jax.ShapeDtypeStruct((2, NPAD, FD), jnp.float32),
        scratch_types=[
            pltpu.VMEM((CH, K), jnp.int32),
            pltpu.VMEM((K,), jnp.float32),
            pltpu.VMEM((K,), jnp.float32),
            pltpu.VMEM((K, FD), jnp.float32),
            pltpu.VMEM((K, FD), jnp.float32),
            pltpu.VMEM_SHARED((NPAD, FD), jnp.float32),
            pltpu.SemaphoreType.DMA,
            pltpu.SemaphoreType.DMA,
            pltpu.SemaphoreType.DMA,
            pltpu.SemaphoreType.DMA,
        ],
    )(_sc_deg_body)
    return fn(ews3, cols3)


def _sumsq_body(x_ref, o_ref):
    blk = x_ref[...]
    o_ref[...] = jnp.sum(blk * blk).reshape(1, 1)


def _sumsq(ew2d):
    n = ew2d.shape[0]
    return pl.pallas_call(
        _sumsq_body,
        out_shape=jax.ShapeDtypeStruct((1, 1), jnp.float32),
        grid=(1,),
        in_specs=[pl.BlockSpec((n, 128), lambda i: (0, 0))],
        out_specs=pl.BlockSpec((1, 1), lambda i: (0, 0)),
    )(ew2d)


def _prep_body(degp_ref, s2_ref, o_ref):
    s = lax.rsqrt(jnp.maximum(s2_ref[...][0, 0], 1e-24))
    d = degp_ref[...]
    deg = s * (d[0] + d[1]) + 1.0
    o_ref[...] = lax.rsqrt(deg)


def _prep(degp, s2):
    blk = 1000
    return pl.pallas_call(
        _prep_body,
        out_shape=jax.ShapeDtypeStruct((N, F), jnp.float32),
        grid=(N // blk,),
        in_specs=[
            pl.BlockSpec((2, blk, FD), lambda i: (0, i, 0)),
            pl.BlockSpec((1, 1), lambda i: (0, 0)),
        ],
        out_specs=pl.BlockSpec((blk, F), lambda i: (i, 0)),
    )(degp, s2)


def _mm1_body(x_ref, w_ref, dis_ref, o_ref):
    xw = jax.lax.dot_general(x_ref[...], w_ref[...], (((1,), (0,)), ((), ())),
                             precision=lax.Precision.HIGHEST,
                             preferred_element_type=jnp.float32)
    o_ref[...] = dis_ref[...] * xw


def _mm1(x, W, dis):
    blk = 1000
    fin = x.shape[1]
    return pl.pallas_call(
        _mm1_body,
        out_shape=jax.ShapeDtypeStruct((N, F), jnp.float32),
        grid=(N // blk,),
        in_specs=[
            pl.BlockSpec((blk, fin), lambda i: (i, 0)),
            pl.BlockSpec((fin, F), lambda i: (0, 0)),
            pl.BlockSpec((blk, F), lambda i: (i, 0)),
        ],
        out_specs=pl.BlockSpec((blk, F), lambda i: (i, 0)),
    )(x, W, dis)


def _layer_body(agg_ref, xwd_ref, dis_ref, s2_ref, b_ref, wn_ref, o_ref):
    s = lax.rsqrt(jnp.maximum(s2_ref[...][0, 0], 1e-24))
    dis = dis_ref[...]
    h = dis * (s * (agg_ref[0] + agg_ref[1]) + xwd_ref[...]) + b_ref[...]
    h = jnp.maximum(h, 0.0)
    hw = jax.lax.dot_general(h, wn_ref[...], (((1,), (0,)), ((), ())),
                             precision=lax.Precision.HIGHEST,
                             preferred_element_type=jnp.float32)
    o_ref[...] = dis * hw


def _layer(agg, xwd, dis, s2, b, Wn):
    blk = 1000
    return pl.pallas_call(
        _layer_body,
        out_shape=jax.ShapeDtypeStruct((N, F), jnp.float32),
        grid=(N // blk,),
        in_specs=[
            pl.BlockSpec((2, blk, F), lambda i: (0, i, 0)),
            pl.BlockSpec((blk, F), lambda i: (i, 0)),
            pl.BlockSpec((blk, F), lambda i: (i, 0)),
            pl.BlockSpec((1, 1), lambda i: (0, 0)),
            pl.BlockSpec((1, F), lambda i: (0, 0)),
            pl.BlockSpec((F, F), lambda i: (0, 0)),
        ],
        out_specs=pl.BlockSpec((blk, F), lambda i: (i, 0)),
    )(agg, xwd, dis, s2, b, Wn)


def _pool_body(agg_ref, xwd_ref, dis_ref, s2_ref, b_ref, batch_ref, o_ref):
    @pl.when(pl.program_id(0) == 0)
    def _():
        o_ref[...] = jnp.zeros_like(o_ref)
    s = lax.rsqrt(jnp.maximum(s2_ref[...][0, 0], 1e-24))
    dis = dis_ref[...]
    h = dis * (s * (agg_ref[0] + agg_ref[1]) + xwd_ref[...]) + b_ref[...]
    h = jnp.maximum(h, 0.0)
    gids = jax.lax.broadcasted_iota(jnp.int32, (1, G), 1)
    onehot = (batch_ref[...] == gids).astype(jnp.float32)
    o_ref[...] += jax.lax.dot_general(
        onehot, h, (((0,), (0,)), ((), ())),
        precision=lax.Precision.HIGHEST,
        preferred_element_type=jnp.float32)


def _pool(agg, xwd, dis, s2, b, batch2d):
    blk = 1000
    return pl.pallas_call(
        _pool_body,
        out_shape=jax.ShapeDtypeStruct((G, F), jnp.float32),
        grid=(N // blk,),
        in_specs=[
            pl.BlockSpec((2, blk, F), lambda i: (0, i, 0)),
            pl.BlockSpec((blk, F), lambda i: (i, 0)),
            pl.BlockSpec((blk, F), lambda i: (i, 0)),
            pl.BlockSpec((1, 1), lambda i: (0, 0)),
            pl.BlockSpec((1, F), lambda i: (0, 0)),
            pl.BlockSpec((blk, 1), lambda i: (i, 0)),
        ],
        out_specs=pl.BlockSpec((G, F), lambda i: (0, 0)),
    )(agg, xwd, dis, s2, b, batch2d)


def _head_body(p_ref, w1_ref, b1_ref, w2_ref, b2_ref, o_ref):
    h1 = jax.lax.dot_general(p_ref[...], w1_ref[...], (((1,), (0,)), ((), ())),
                             precision=lax.Precision.HIGHEST,
                             preferred_element_type=jnp.float32)
    h1 = jnp.maximum(h1 + b1_ref[...], 0.0)
    t = jax.lax.dot_general(h1, w2_ref[...], (((1,), (0,)), ((), ())),
                            precision=lax.Precision.HIGHEST,
                            preferred_element_type=jnp.float32) + b2_ref[...]
    m = jnp.max(t, axis=-1, keepdims=True)
    lse = jnp.log(jnp.sum(jnp.exp(t - m), axis=-1, keepdims=True)) + m
    o_ref[...] = t - lse


def _head(pooled, Wl1, bl1, Wl2p, bl2p):
    return pl.pallas_call(
        _head_body,
        out_shape=jax.ShapeDtypeStruct((G, F), jnp.float32),
        in_specs=[pl.BlockSpec(a.shape, lambda: tuple(0 for _ in a.shape))
                  for a in (pooled, Wl1, bl1, Wl2p, bl2p)],
        out_specs=pl.BlockSpec((G, F), lambda: (0, 0)),
    )(pooled, Wl1, bl1, Wl2p, bl2p)


def kernel(x, edge_index, edge_weight, batch,
           W1, b1, W2, b2, W3, b3, W4, b4, W5, b5, Wl1, bl1, Wl2, bl2):
    pad = TILES * PERP - E
    spread = jnp.arange(pad, dtype=jnp.int32)
    row = jnp.concatenate([edge_index[0], spread % N])
    col = jnp.concatenate([edge_index[1], spread % NPAD])
    ewp = jnp.concatenate([edge_weight, jnp.zeros((pad,), jnp.float32)])
    rows3 = row.reshape(TILES, CH, K)
    cols3 = col.reshape(TILES, CH, K)
    ews3 = ewp.reshape(TILES, CH, K)

    s2 = _sumsq(edge_weight.reshape(2500, 128))
    degp = _sc_deg(ews3, cols3)
    dis = _prep(degp, s2)

    batch2d = batch.reshape(N, 1)
    bs = [b1, b2, b3, b4, b5]
    Ws = [W2, W3, W4, W5]

    xwd = _mm1(x, W1, dis)
    for l in range(4):
        agg = _sc_agg(xwd, rows3, ews3, cols3)
        xwd = _layer(agg, xwd, dis, s2, bs[l].reshape(1, F), Ws[l])
    agg = _sc_agg(xwd, rows3, ews3, cols3)
    pooled = _pool(agg, xwd, dis, s2, bs[4].reshape(1, F), batch2d)

    Wl2p = jnp.zeros((F, F), jnp.float32).at[:, :C].set(Wl2)
    bl2p = jnp.full((1, F), -1e30, jnp.float32).at[0, :C].set(bl2)
    out = _head(pooled, Wl1, bl1.reshape(1, F), Wl2p, bl2p)
    return out[:, :C]


# speedup vs baseline: 1.0011x; 1.0011x over previous
import functools

import jax
import jax.numpy as jnp
from jax import lax
from jax.experimental import pallas as pl
from jax.experimental.pallas import tpu as pltpu
from jax.experimental.pallas import tpu_sc as plsc

N = 10000
E = 320000
F = 128
C = 32
G = 128

TILES = 32
K = 128
PER = E // TILES
CH = 80
PERP = CH * K
SCH = CH // 2
NPAD = 10240
RPT = NPAD // 16
LANES = 16


def _zero_acc(buf, acc_sh, base):
    def zrow(i, _):
        for j in range(8):
            buf[i, pl.ds(j * LANES, LANES)] = jnp.zeros((LANES,), jnp.float32)
        return 0
    lax.fori_loop(0, K, zrow, 0)
    nfull, rem = RPT // K, RPT % K
    for i in range(nfull):
        pltpu.sync_copy(buf, acc_sh.at[pl.ds(base + i * K, K)])
    if rem:
        pltpu.sync_copy(buf.at[pl.ds(0, rem)],
                        acc_sh.at[pl.ds(base + nfull * K, rem)])


def _sc_agg_body(xwd_hbm, rows_hbm, ews_hbm, cols_hbm, out_hbm,
                 cidx_all, ri_a, ri_b, ew_a, ew_b, buf_a, buf_b, acc_sh,
                 sem_ga, sem_gb, sem_sa, sem_sb, sem_ia, sem_ib):
    c = lax.axis_index("c")
    s = lax.axis_index("s")
    base = s * RPT
    w = c * 16 + s

    _zero_acc(buf_a, acc_sh, base)
    pltpu.sync_copy(cols_hbm.at[w], cidx_all)
    plsc.subcore_barrier()

    def fire_idx(ci, ri, ew, sem):
        pltpu.async_copy(rows_hbm.at[w, ci], ri, sem)
        pltpu.async_copy(ews_hbm.at[w, ci], ew, sem)

    def drain_idx(ci, ri, ew, sem):
        pltpu.make_async_copy(rows_hbm.at[w, ci], ri, sem).wait()
        pltpu.make_async_copy(ews_hbm.at[w, ci], ew, sem).wait()

    def fire_gather(ri, buf, sem):
        pltpu.async_copy(xwd_hbm.at[ri], buf, sem)

    def drain_gather(ri, buf, sem):
        pltpu.make_async_copy(xwd_hbm.at[ri], buf, sem).wait()

    def fire_scatter(ci, buf, sem):
        pltpu.async_copy(buf, acc_sh.at[cidx_all.at[ci]], sem, add=True)

    def drain_scatter(ci, buf, sem):
        pltpu.make_async_copy(buf, acc_sh.at[cidx_all.at[ci]], sem).wait()

    def scale(buf, ew):
        def grp(g, _):
            off = g * LANES
            ev = ew[pl.ds(off, LANES)]
            for j in range(LANES):
                e = ev[j]
                for jc in range(8):
                    sl = pl.ds(jc * LANES, LANES)
                    buf[off + j, sl] = buf[off + j, sl] * e
            return 0
        lax.fori_loop(0, K // LANES, grp, 0)

    fire_idx(0, ri_a, ew_a, sem_ia)
    fire_idx(1, ri_b, ew_b, sem_ib)
    drain_idx(0, ri_a, ew_a, sem_ia)
    fire_gather(ri_a, buf_a, sem_ga)

    def body(i, _):
        c0 = 2 * i
        c1 = 2 * i + 1
        drain_gather(ri_a, buf_a, sem_ga)
        scale(buf_a, ew_a)

        @pl.when(i > 0)
        def _():
            drain_scatter(c1, buf_b, sem_sb)
        fire_scatter(c0, buf_a, sem_sa)
        drain_idx(c1, ri_b, ew_b, sem_ib)
        fire_gather(ri_b, buf_b, sem_gb)

        @pl.when(i < SCH - 1)
        def _():
            fire_idx(c0 + 2, ri_a, ew_a, sem_ia)
        drain_gather(ri_b, buf_b, sem_gb)
        scale(buf_b, ew_b)
        drain_scatter(c0, buf_a, sem_sa)
        fire_scatter(c1, buf_b, sem_sb)

        @pl.when(i < SCH - 1)
        def _():
            drain_idx(c0 + 2, ri_a, ew_a, sem_ia)
            fire_gather(ri_a, buf_a, sem_ga)
            fire_idx(c1 + 2, ri_b, ew_b, sem_ib)
        return 0
    lax.fori_loop(0, SCH, body, 0)
    drain_scatter(CH - 1, buf_b, sem_sb)

    plsc.subcore_barrier()
    pltpu.sync_copy(acc_sh.at[pl.ds(base, RPT)],
                    out_hbm.at[c, pl.ds(base, RPT)])


def _sc_agg(xwd, rows3, ews3, cols3):
    mesh = plsc.VectorSubcoreMesh(core_axis_name="c", subcore_axis_name="s")
    fn = functools.partial(
        pl.kernel, mesh=mesh,
        out_type=jax.ShapeDtypeStruct((2, NPAD, F), jnp.float32),
        scratch_types=[
            pltpu.VMEM((CH, K), jnp.int32),
            pltpu.VMEM((K,), jnp.int32),
            pltpu.VMEM((K,), jnp.int32),
            pltpu.VMEM((K,), jnp.float32),
            pltpu.VMEM((K,), jnp.float32),
            pltpu.VMEM((K, F), jnp.float32),
            pltpu.VMEM((K, F), jnp.float32),
            pltpu.VMEM_SHARED((NPAD, F), jnp.float32),
            pltpu.SemaphoreType.DMA,
            pltpu.SemaphoreType.DMA,
            pltpu.SemaphoreType.DMA,
            pltpu.SemaphoreType.DMA,
            pltpu.SemaphoreType.DMA,
            pltpu.SemaphoreType.DMA,
        ],
    )(_sc_agg_body)
    return fn(xwd, rows3, ews3, cols3)


FD = F


def _sc_deg_body(ews_hbm, cols_hbm, out_hbm,
                 cidx_all, ew_a, ew_b, buf_a, buf_b, acc_sh,
                 sem_sa, sem_sb, sem_ia, sem_ib):
    c = lax.axis_index("c")
    s = lax.axis_index("s")
    base = s * RPT
    w = c * 16 + s

    _zero_acc(buf_a, acc_sh, base)
    pltpu.sync_copy(cols_hbm.at[w], cidx_all)
    plsc.subcore_barrier()

    def fire_idx(ci, ew, sem):
        pltpu.async_copy(ews_hbm.at[w, ci], ew, sem)

    def drain_idx(ci, ew, sem):
        pltpu.make_async_copy(ews_hbm.at[w, ci], ew, sem).wait()

    def fire_scatter(ci, buf, sem):
        pltpu.async_copy(buf, acc_sh.at[cidx_all.at[ci]], sem, add=True)

    def drain_scatter(ci, buf, sem):
        pltpu.make_async_copy(buf, acc_sh.at[cidx_all.at[ci]], sem).wait()

    def build(buf, ew):
        def grp(g, _):
            off = g * LANES
            ev = ew[pl.ds(off, LANES)]
            for j in range(LANES):
                e16 = jnp.full((LANES,), ev[j], jnp.float32)
                for jc in range(FD // LANES):
                    buf[off + j, pl.ds(jc * LANES, LANES)] = e16
            return 0
        lax.fori_loop(0, K // LANES, grp, 0)

    fire_idx(0, ew_a, sem_ia)
    fire_idx(1, ew_b, sem_ib)

    def body(i, _):
        c0 = 2 * i
        c1 = 2 * i + 1
        drain_idx(c0, ew_a, sem_ia)
        build(buf_a, ew_a)

        @pl.when(i > 0)
        def _():
            drain_scatter(c1, buf_b, sem_sb)
        fire_scatter(c0, buf_a, sem_sa)
        drain_idx(c1, ew_b, sem_ib)
        build(buf_b, ew_b)
        drain_scatter(c0, buf_a, sem_sa)
        fire_scatter(c1, buf_b, sem_sb)

        @pl.when(i < SCH - 1)
        def _():
            fire_idx(c0 + 2, ew_a, sem_ia)
            fire_idx(c1 + 2, ew_b, sem_ib)
        return 0
    lax.fori_loop(0, SCH, body, 0)
    drain_scatter(CH - 1, buf_b, sem_sb)

    plsc.subcore_barrier()
    pltpu.sync_copy(acc_sh.at[pl.ds(base, RPT)],
                    out_hbm.at[c, pl.ds(base, RPT)])


def _sc_deg(ews3, cols3):
    mesh = plsc.VectorSubcoreMesh(core_axis_name="c", subcore_axis_name="s")
    fn = functools.partial(
        pl.kernel, mesh=mesh,
        out_type=jax.ShapeDtypeStruct((2, NPAD, FD), jnp.float32),
        scratch_types=[
            pltpu.VMEM((CH, K), jnp.int32),
            pltpu.VMEM((K,), jnp.float32),
            pltpu.VMEM((K,), jnp.float32),
            pltpu.VMEM((K, FD), jnp.float32),
            pltpu.VMEM((K, FD), jnp.float32),
            pltpu.VMEM_SHARED((NPAD, FD), jnp.float32),
            pltpu.SemaphoreType.DMA,
            pltpu.SemaphoreType.DMA,
            pltpu.SemaphoreType.DMA,
            pltpu.SemaphoreType.DMA,
        ],
    )(_sc_deg_body)
    return fn(ews3, cols3)


def _sumsq_body(x_ref, o_ref):
    blk = x_ref[...]
    o_ref[...] = jnp.sum(blk * blk).reshape(1, 1)


def _sumsq(ew2d):
    n = ew2d.shape[0]
    return pl.pallas_call(
        _sumsq_body,
        out_shape=jax.ShapeDtypeStruct((1, 1), jnp.float32),
        grid=(1,),
        in_specs=[pl.BlockSpec((n, 128), lambda i: (0, 0))],
        out_specs=pl.BlockSpec((1, 1), lambda i: (0, 0)),
    )(ew2d)


def _prep_body(degp_ref, s2_ref, o_ref):
    s = lax.rsqrt(jnp.maximum(s2_ref[...][0, 0], 1e-24))
    d = degp_ref[...]
    deg = s * (d[0] + d[1]) + 1.0
    o_ref[...] = lax.rsqrt(deg)


def _prep(degp, s2):
    blk = 1000
    return pl.pallas_call(
        _prep_body,
        out_shape=jax.ShapeDtypeStruct((N, F), jnp.float32),
        grid=(N // blk,),
        in_specs=[
            pl.BlockSpec((2, blk, FD), lambda i: (0, i, 0)),
            pl.BlockSpec((1, 1), lambda i: (0, 0)),
        ],
        out_specs=pl.BlockSpec((blk, F), lambda i: (i, 0)),
    )(degp, s2)


def _mm1_body(x_ref, w_ref, dis_ref, o_ref):
    xw = jax.lax.dot_general(x_ref[...], w_ref[...], (((1,), (0,)), ((), ())),
                             precision=lax.Precision.HIGHEST,
                             preferred_element_type=jnp.float32)
    o_ref[...] = dis_ref[...] * xw


def _mm1(x, W, dis):
    blk = 1000
    fin = x.shape[1]
    return pl.pallas_call(
        _mm1_body,
        out_shape=jax.ShapeDtypeStruct((N, F), jnp.float32),
        grid=(N // blk,),
        in_specs=[
            pl.BlockSpec((blk, fin), lambda i: (i, 0)),
            pl.BlockSpec((fin, F), lambda i: (0, 0)),
            pl.BlockSpec((blk, F), lambda i: (i, 0)),
        ],
        out_specs=pl.BlockSpec((blk, F), lambda i: (i, 0)),
    )(x, W, dis)


def _layer_body(agg_ref, xwd_ref, dis_ref, s2_ref, b_ref, wn_ref, o_ref):
    s = lax.rsqrt(jnp.maximum(s2_ref[...][0, 0], 1e-24))
    dis = dis_ref[...]
    h = dis * (s * (agg_ref[0] + agg_ref[1]) + xwd_ref[...]) + b_ref[...]
    h = jnp.maximum(h, 0.0)
    hw = jax.lax.dot_general(h, wn_ref[...], (((1,), (0,)), ((), ())),
                             precision=lax.Precision.HIGHEST,
                             preferred_element_type=jnp.float32)
    o_ref[...] = dis * hw


def _layer(agg, xwd, dis, s2, b, Wn):
    blk = 1000
    return pl.pallas_call(
        _layer_body,
        out_shape=jax.ShapeDtypeStruct((N, F), jnp.float32),
        grid=(N // blk,),
        in_specs=[
            pl.BlockSpec((2, blk, F), lambda i: (0, i, 0)),
            pl.BlockSpec((blk, F), lambda i: (i, 0)),
            pl.BlockSpec((blk, F), lambda i: (i, 0)),
            pl.BlockSpec((1, 1), lambda i: (0, 0)),
            pl.BlockSpec((1, F), lambda i: (0, 0)),
            pl.BlockSpec((F, F), lambda i: (0, 0)),
        ],
        out_specs=pl.BlockSpec((blk, F), lambda i: (i, 0)),
    )(agg, xwd, dis, s2, b, Wn)


def _pool_body(agg_ref, xwd_ref, dis_ref, s2_ref, b_ref, batch_ref, o_ref):
    @pl.when(pl.program_id(0) == 0)
    def _():
        o_ref[...] = jnp.zeros_like(o_ref)
    s = lax.rsqrt(jnp.maximum(s2_ref[...][0, 0], 1e-24))
    dis = dis_ref[...]
    h = dis * (s * (agg_ref[0] + agg_ref[1]) + xwd_ref[...]) + b_ref[...]
    h = jnp.maximum(h, 0.0)
    gids = jax.lax.broadcasted_iota(jnp.int32, (1, G), 1)
    onehot = (batch_ref[...] == gids).astype(jnp.float32)
    o_ref[...] += jax.lax.dot_general(
        onehot, h, (((0,), (0,)), ((), ())),
        precision=lax.Precision.HIGHEST,
        preferred_element_type=jnp.float32)


def _pool(agg, xwd, dis, s2, b, batch2d):
    blk = 1000
    return pl.pallas_call(
        _pool_body,
        out_shape=jax.ShapeDtypeStruct((G, F), jnp.float32),
        grid=(N // blk,),
        in_specs=[
            pl.BlockSpec((2, blk, F), lambda i: (0, i, 0)),
            pl.BlockSpec((blk, F), lambda i: (i, 0)),
            pl.BlockSpec((blk, F), lambda i: (i, 0)),
            pl.BlockSpec((1, 1), lambda i: (0, 0)),
            pl.BlockSpec((1, F), lambda i: (0, 0)),
            pl.BlockSpec((blk, 1), lambda i: (i, 0)),
        ],
        out_specs=pl.BlockSpec((G, F), lambda i: (0, 0)),
    )(agg, xwd, dis, s2, b, batch2d)


def _head_body(p_ref, w1_ref, b1_ref, w2_ref, b2_ref, o_ref):
    h1 = jax.lax.dot_general(p_ref[...], w1_ref[...], (((1,), (0,)), ((), ())),
                             precision=lax.Precision.HIGHEST,
                             preferred_element_type=jnp.float32)
    h1 = jnp.maximum(h1 + b1_ref[...], 0.0)
    t = jax.lax.dot_general(h1, w2_ref[...], (((1,), (0,)), ((), ())),
                            precision=lax.Precision.HIGHEST,
                            preferred_element_type=jnp.float32) + b2_ref[...]
    m = jnp.max(t, axis=-1, keepdims=True)
    lse = jnp.log(jnp.sum(jnp.exp(t - m), axis=-1, keepdims=True)) + m
    o_ref[...] = t - lse


def _head(pooled, Wl1, bl1, Wl2p, bl2p):
    return pl.pallas_call(
        _head_body,
        out_shape=jax.ShapeDtypeStruct((G, F), jnp.float32),
        in_specs=[pl.BlockSpec(a.shape, lambda: tuple(0 for _ in a.shape))
                  for a in (pooled, Wl1, bl1, Wl2p, bl2p)],
        out_specs=pl.BlockSpec((G, F), lambda: (0, 0)),
    )(pooled, Wl1, bl1, Wl2p, bl2p)


def kernel(x, edge_index, edge_weight, batch,
           W1, b1, W2, b2, W3, b3, W4, b4, W5, b5, Wl1, bl1, Wl2, bl2):
    pad = TILES * PERP - E
    spread = jnp.arange(pad, dtype=jnp.int32)
    row = jnp.concatenate([edge_index[0], spread % N])
    col = jnp.concatenate([edge_index[1], spread % NPAD])
    ewp = jnp.concatenate([edge_weight, jnp.zeros((pad,), jnp.float32)])
    rows3 = row.reshape(TILES, CH, K)
    cols3 = col.reshape(TILES, CH, K)
    ews3 = ewp.reshape(TILES, CH, K)

    s2 = _sumsq(edge_weight.reshape(2500, 128))
    degp = _sc_deg(ews3, cols3)
    dis = _prep(degp, s2)

    batch2d = batch.reshape(N, 1)
    bs = [b1, b2, b3, b4, b5]
    Ws = [W2, W3, W4, W5]

    xwd = _mm1(x, W1, dis)
    for l in range(4):
        agg = _sc_agg(xwd, rows3, ews3, cols3)
        xwd = _layer(agg, xwd, dis, s2, bs[l].reshape(1, F), Ws[l])
    agg = _sc_agg(xwd, rows3, ews3, cols3)
    pooled = _pool(agg, xwd, dis, s2, bs[4].reshape(1, F), batch2d)

    Wl2p = jnp.zeros((F, F), jnp.float32).at[:, :C].set(Wl2)
    bl2p = jnp.full((1, F), -1e30, jnp.float32).at[0, :C].set(bl2)
    out = _head(pooled, Wl1, bl1.reshape(1, F), Wl2p, bl2p)
    return out[:, :C]


# speedup vs baseline: 1.0250x; 1.0239x over previous
import functools

import jax
import jax.numpy as jnp
from jax import lax
from jax.experimental import pallas as pl
from jax.experimental.pallas import tpu as pltpu
from jax.experimental.pallas import tpu_sc as plsc

N = 10000
E = 320000
F = 128
C = 32
G = 128

TILES = 32
K = 128
PER = E // TILES
CH = 80
PERP = CH * K
SCH = CH // 2
NPAD = 10240
RPT = NPAD // 16
LANES = 16


def _zero_acc(buf, acc_sh, base):
    def zrow(i, _):
        for j in range(8):
            buf[i, pl.ds(j * LANES, LANES)] = jnp.zeros((LANES,), jnp.float32)
        return 0
    lax.fori_loop(0, K, zrow, 0)
    nfull, rem = RPT // K, RPT % K
    for i in range(nfull):
        pltpu.sync_copy(buf, acc_sh.at[pl.ds(base + i * K, K)])
    if rem:
        pltpu.sync_copy(buf.at[pl.ds(0, rem)],
                        acc_sh.at[pl.ds(base + nfull * K, rem)])


def _sc_agg_body(xwd_hbm, rows_hbm, ews_hbm, cols_hbm, out_hbm,
                 cidx_all, ri_a, ri_b, ew_a, ew_b, buf_a, buf_b, acc_sh,
                 sem_ga, sem_gb, sem_sa, sem_sb, sem_ia, sem_ib):
    c = lax.axis_index("c")
    s = lax.axis_index("s")
    base = s * RPT
    w = c * 16 + s

    _zero_acc(buf_a, acc_sh, base)
    pltpu.sync_copy(cols_hbm.at[w], cidx_all)
    plsc.subcore_barrier()

    def fire_idx(ci, ri, ew, sem):
        pltpu.async_copy(rows_hbm.at[w, ci], ri, sem)
        pltpu.async_copy(ews_hbm.at[w, ci], ew, sem)

    def drain_idx(ci, ri, ew, sem):
        pltpu.make_async_copy(rows_hbm.at[w, ci], ri, sem).wait()
        pltpu.make_async_copy(ews_hbm.at[w, ci], ew, sem).wait()

    def fire_gather(ri, buf, sem):
        pltpu.async_copy(xwd_hbm.at[ri], buf, sem)

    def drain_gather(ri, buf, sem):
        pltpu.make_async_copy(xwd_hbm.at[ri], buf, sem).wait()

    def fire_scatter(ci, buf, sem):
        pltpu.async_copy(buf, acc_sh.at[cidx_all.at[ci]], sem, add=True)

    def drain_scatter(ci, buf, sem):
        pltpu.make_async_copy(buf, acc_sh.at[cidx_all.at[ci]], sem).wait()

    def scale(buf, ew):
        def grp(g, _):
            off = g * LANES
            ev = ew[pl.ds(off, LANES)]
            for j in range(LANES):
                e = ev[j]
                for jc in range(8):
                    sl = pl.ds(jc * LANES, LANES)
                    buf[off + j, sl] = buf[off + j, sl] * e
            return 0
        lax.fori_loop(0, K // LANES, grp, 0)

    fire_idx(0, ri_a, ew_a, sem_ia)
    fire_idx(1, ri_b, ew_b, sem_ib)
    drain_idx(0, ri_a, ew_a, sem_ia)
    fire_gather(ri_a, buf_a, sem_ga)

    def body(i, _):
        c0 = 2 * i
        c1 = 2 * i + 1
        drain_gather(ri_a, buf_a, sem_ga)
        scale(buf_a, ew_a)

        @pl.when(i > 0)
        def _():
            drain_scatter(c1, buf_b, sem_sb)
        fire_scatter(c0, buf_a, sem_sa)
        drain_idx(c1, ri_b, ew_b, sem_ib)
        fire_gather(ri_b, buf_b, sem_gb)

        @pl.when(i < SCH - 1)
        def _():
            fire_idx(c0 + 2, ri_a, ew_a, sem_ia)
        drain_gather(ri_b, buf_b, sem_gb)
        scale(buf_b, ew_b)
        drain_scatter(c0, buf_a, sem_sa)
        fire_scatter(c1, buf_b, sem_sb)

        @pl.when(i < SCH - 1)
        def _():
            drain_idx(c0 + 2, ri_a, ew_a, sem_ia)
            fire_gather(ri_a, buf_a, sem_ga)
            fire_idx(c1 + 2, ri_b, ew_b, sem_ib)
        return 0
    lax.fori_loop(0, SCH, body, 0)
    drain_scatter(CH - 1, buf_b, sem_sb)

    plsc.subcore_barrier()
    pltpu.sync_copy(acc_sh.at[pl.ds(base, RPT)],
                    out_hbm.at[c, pl.ds(base, RPT)])


def _sc_agg(xwd, rows3, ews3, cols3):
    mesh = plsc.VectorSubcoreMesh(core_axis_name="c", subcore_axis_name="s")
    fn = functools.partial(
        pl.kernel, mesh=mesh,
        out_type=jax.ShapeDtypeStruct((2, NPAD, F), jnp.float32),
        scratch_types=[
            pltpu.VMEM((CH, K), jnp.int32),
            pltpu.VMEM((K,), jnp.int32),
            pltpu.VMEM((K,), jnp.int32),
            pltpu.VMEM((K,), jnp.float32),
            pltpu.VMEM((K,), jnp.float32),
            pltpu.VMEM((K, F), jnp.float32),
            pltpu.VMEM((K, F), jnp.float32),
            pltpu.VMEM_SHARED((NPAD, F), jnp.float32),
            pltpu.SemaphoreType.DMA,
            pltpu.SemaphoreType.DMA,
            pltpu.SemaphoreType.DMA,
            pltpu.SemaphoreType.DMA,
            pltpu.SemaphoreType.DMA,
            pltpu.SemaphoreType.DMA,
        ],
    )(_sc_agg_body)
    return fn(xwd, rows3, ews3, cols3)


FD = F


def _sc_deg_body(ews_hbm, cols_hbm, out_hbm,
                 cidx_all, ew_a, ew_b, buf_a, buf_b, acc_sh,
                 sem_sa, sem_sb, sem_ia, sem_ib):
    c = lax.axis_index("c")
    s = lax.axis_index("s")
    base = s * RPT
    w = c * 16 + s

    _zero_acc(buf_a, acc_sh, base)
    pltpu.sync_copy(cols_hbm.at[w], cidx_all)
    plsc.subcore_barrier()

    def fire_idx(ci, ew, sem):
        pltpu.async_copy(ews_hbm.at[w, ci], ew, sem)

    def drain_idx(ci, ew, sem):
        pltpu.make_async_copy(ews_hbm.at[w, ci], ew, sem).wait()

    def fire_scatter(ci, buf, sem):
        pltpu.async_copy(buf, acc_sh.at[cidx_all.at[ci]], sem, add=True)

    def drain_scatter(ci, buf, sem):
        pltpu.make_async_copy(buf, acc_sh.at[cidx_all.at[ci]], sem).wait()

    def build(buf, ew):
        def grp(g, _):
            off = g * LANES
            ev = ew[pl.ds(off, LANES)]
            for j in range(LANES):
                e16 = jnp.full((LANES,), ev[j], jnp.float32)
                for jc in range(FD // LANES):
                    buf[off + j, pl.ds(jc * LANES, LANES)] = e16
            return 0
        lax.fori_loop(0, K // LANES, grp, 0)

    fire_idx(0, ew_a, sem_ia)
    fire_idx(1, ew_b, sem_ib)

    def body(i, _):
        c0 = 2 * i
        c1 = 2 * i + 1
        drain_idx(c0, ew_a, sem_ia)
        build(buf_a, ew_a)

        @pl.when(i > 0)
        def _():
            drain_scatter(c1, buf_b, sem_sb)
        fire_scatter(c0, buf_a, sem_sa)
        drain_idx(c1, ew_b, sem_ib)
        build(buf_b, ew_b)
        drain_scatter(c0, buf_a, sem_sa)
        fire_scatter(c1, buf_b, sem_sb)

        @pl.when(i < SCH - 1)
        def _():
            fire_idx(c0 + 2, ew_a, sem_ia)
            fire_idx(c1 + 2, ew_b, sem_ib)
        return 0
    lax.fori_loop(0, SCH, body, 0)
    drain_scatter(CH - 1, buf_b, sem_sb)

    plsc.subcore_barrier()
    pltpu.sync_copy(acc_sh.at[pl.ds(base, RPT)],
                    out_hbm.at[c, pl.ds(base, RPT)])


def _sc_deg(ews3, cols3):
    mesh = plsc.VectorSubcoreMesh(core_axis_name="c", subcore_axis_name="s")
    fn = functools.partial(
        pl.kernel, mesh=mesh,
        out_type=jax.ShapeDtypeStruct((2, NPAD, FD), jnp.float32),
        scratch_types=[
            pltpu.VMEM((CH, K), jnp.int32),
            pltpu.VMEM((K,), jnp.float32),
            pltpu.VMEM((K,), jnp.float32),
            pltpu.VMEM((K, FD), jnp.float32),
            pltpu.VMEM((K, FD), jnp.float32),
            pltpu.VMEM_SHARED((NPAD, FD), jnp.float32),
            pltpu.SemaphoreType.DMA,
            pltpu.SemaphoreType.DMA,
            pltpu.SemaphoreType.DMA,
            pltpu.SemaphoreType.DMA,
        ],
    )(_sc_deg_body)
    return fn(ews3, cols3)


def _sumsq_body(x_ref, o_ref):
    blk = x_ref[...]
    o_ref[...] = jnp.sum(blk * blk).reshape(1, 1)


def _sumsq(ew2d):
    n = ew2d.shape[0]
    return pl.pallas_call(
        _sumsq_body,
        out_shape=jax.ShapeDtypeStruct((1, 1), jnp.float32),
        grid=(1,),
        in_specs=[pl.BlockSpec((n, 128), lambda i: (0, 0))],
        out_specs=pl.BlockSpec((1, 1), lambda i: (0, 0)),
    )(ew2d)


def _prep_body(degp_ref, s2_ref, o_ref):
    s = lax.rsqrt(jnp.maximum(s2_ref[...][0, 0], 1e-24))
    d = degp_ref[...]
    deg = s * (d[0] + d[1]) + 1.0
    o_ref[...] = lax.rsqrt(deg)


def _prep(degp, s2):
    blk = 2000
    return pl.pallas_call(
        _prep_body,
        out_shape=jax.ShapeDtypeStruct((N, F), jnp.float32),
        grid=(N // blk,),
        in_specs=[
            pl.BlockSpec((2, blk, FD), lambda i: (0, i, 0)),
            pl.BlockSpec((1, 1), lambda i: (0, 0)),
        ],
        out_specs=pl.BlockSpec((blk, F), lambda i: (i, 0)),
    )(degp, s2)


def _mm1_body(x_ref, w_ref, dis_ref, o_ref):
    xw = jax.lax.dot_general(x_ref[...], w_ref[...], (((1,), (0,)), ((), ())),
                             precision=lax.Precision.HIGHEST,
                             preferred_element_type=jnp.float32)
    o_ref[...] = dis_ref[...] * xw


def _mm1(x, W, dis):
    blk = 2000
    fin = x.shape[1]
    return pl.pallas_call(
        _mm1_body,
        out_shape=jax.ShapeDtypeStruct((N, F), jnp.float32),
        grid=(N // blk,),
        in_specs=[
            pl.BlockSpec((blk, fin), lambda i: (i, 0)),
            pl.BlockSpec((fin, F), lambda i: (0, 0)),
            pl.BlockSpec((blk, F), lambda i: (i, 0)),
        ],
        out_specs=pl.BlockSpec((blk, F), lambda i: (i, 0)),
    )(x, W, dis)


def _layer_body(agg_ref, xwd_ref, dis_ref, s2_ref, b_ref, wn_ref, o_ref):
    s = lax.rsqrt(jnp.maximum(s2_ref[...][0, 0], 1e-24))
    dis = dis_ref[...]
    h = dis * (s * (agg_ref[0] + agg_ref[1]) + xwd_ref[...]) + b_ref[...]
    h = jnp.maximum(h, 0.0)
    hw = jax.lax.dot_general(h, wn_ref[...], (((1,), (0,)), ((), ())),
                             precision=lax.Precision.HIGHEST,
                             preferred_element_type=jnp.float32)
    o_ref[...] = dis * hw


def _layer(agg, xwd, dis, s2, b, Wn):
    blk = 2000
    return pl.pallas_call(
        _layer_body,
        out_shape=jax.ShapeDtypeStruct((N, F), jnp.float32),
        grid=(N // blk,),
        in_specs=[
            pl.BlockSpec((2, blk, F), lambda i: (0, i, 0)),
            pl.BlockSpec((blk, F), lambda i: (i, 0)),
            pl.BlockSpec((blk, F), lambda i: (i, 0)),
            pl.BlockSpec((1, 1), lambda i: (0, 0)),
            pl.BlockSpec((1, F), lambda i: (0, 0)),
            pl.BlockSpec((F, F), lambda i: (0, 0)),
        ],
        out_specs=pl.BlockSpec((blk, F), lambda i: (i, 0)),
    )(agg, xwd, dis, s2, b, Wn)


def _pool_body(agg_ref, xwd_ref, dis_ref, s2_ref, b_ref, batch_ref, o_ref):
    @pl.when(pl.program_id(0) == 0)
    def _():
        o_ref[...] = jnp.zeros_like(o_ref)
    s = lax.rsqrt(jnp.maximum(s2_ref[...][0, 0], 1e-24))
    dis = dis_ref[...]
    h = dis * (s * (agg_ref[0] + agg_ref[1]) + xwd_ref[...]) + b_ref[...]
    h = jnp.maximum(h, 0.0)
    gids = jax.lax.broadcasted_iota(jnp.int32, (1, G), 1)
    onehot = (batch_ref[...] == gids).astype(jnp.float32)
    o_ref[...] += jax.lax.dot_general(
        onehot, h, (((0,), (0,)), ((), ())),
        precision=lax.Precision.HIGHEST,
        preferred_element_type=jnp.float32)


def _pool(agg, xwd, dis, s2, b, batch2d):
    blk = 2000
    return pl.pallas_call(
        _pool_body,
        out_shape=jax.ShapeDtypeStruct((G, F), jnp.float32),
        grid=(N // blk,),
        in_specs=[
            pl.BlockSpec((2, blk, F), lambda i: (0, i, 0)),
            pl.BlockSpec((blk, F), lambda i: (i, 0)),
            pl.BlockSpec((blk, F), lambda i: (i, 0)),
            pl.BlockSpec((1, 1), lambda i: (0, 0)),
            pl.BlockSpec((1, F), lambda i: (0, 0)),
            pl.BlockSpec((blk, 1), lambda i: (i, 0)),
        ],
        out_specs=pl.BlockSpec((G, F), lambda i: (0, 0)),
    )(agg, xwd, dis, s2, b, batch2d)


def _head_body(p_ref, w1_ref, b1_ref, w2_ref, b2_ref, o_ref):
    h1 = jax.lax.dot_general(p_ref[...], w1_ref[...], (((1,), (0,)), ((), ())),
                             precision=lax.Precision.HIGHEST,
                             preferred_element_type=jnp.float32)
    h1 = jnp.maximum(h1 + b1_ref[...], 0.0)
    t = jax.lax.dot_general(h1, w2_ref[...], (((1,), (0,)), ((), ())),
                            precision=lax.Precision.HIGHEST,
                            preferred_element_type=jnp.float32) + b2_ref[...]
    m = jnp.max(t, axis=-1, keepdims=True)
    lse = jnp.log(jnp.sum(jnp.exp(t - m), axis=-1, keepdims=True)) + m
    o_ref[...] = t - lse


def _head(pooled, Wl1, bl1, Wl2p, bl2p):
    return pl.pallas_call(
        _head_body,
        out_shape=jax.ShapeDtypeStruct((G, F), jnp.float32),
        in_specs=[pl.BlockSpec(a.shape, lambda: tuple(0 for _ in a.shape))
                  for a in (pooled, Wl1, bl1, Wl2p, bl2p)],
        out_specs=pl.BlockSpec((G, F), lambda: (0, 0)),
    )(pooled, Wl1, bl1, Wl2p, bl2p)


def kernel(x, edge_index, edge_weight, batch,
           W1, b1, W2, b2, W3, b3, W4, b4, W5, b5, Wl1, bl1, Wl2, bl2):
    pad = TILES * PERP - E
    spread = jnp.arange(pad, dtype=jnp.int32)
    row = jnp.concatenate([edge_index[0], spread % N])
    col = jnp.concatenate([edge_index[1], spread % NPAD])
    ewp = jnp.concatenate([edge_weight, jnp.zeros((pad,), jnp.float32)])
    rows3 = row.reshape(TILES, CH, K)
    cols3 = col.reshape(TILES, CH, K)
    ews3 = ewp.reshape(TILES, CH, K)

    s2 = _sumsq(edge_weight.reshape(2500, 128))
    degp = _sc_deg(ews3, cols3)
    dis = _prep(degp, s2)

    batch2d = batch.reshape(N, 1)
    bs = [b1, b2, b3, b4, b5]
    Ws = [W2, W3, W4, W5]

    xwd = _mm1(x, W1, dis)
    for l in range(4):
        agg = _sc_agg(xwd, rows3, ews3, cols3)
        xwd = _layer(agg, xwd, dis, s2, bs[l].reshape(1, F), Ws[l])
    agg = _sc_agg(xwd, rows3, ews3, cols3)
    pooled = _pool(agg, xwd, dis, s2, bs[4].reshape(1, F), batch2d)

    Wl2p = jnp.zeros((F, F), jnp.float32).at[:, :C].set(Wl2)
    bl2p = jnp.full((1, F), -1e30, jnp.float32).at[0, :C].set(bl2)
    out = _head(pooled, Wl1, bl1.reshape(1, F), Wl2p, bl2p)
    return out[:, :C]


# speedup vs baseline: 1.0284x; 1.0033x over previous
import functools

import jax
import jax.numpy as jnp
from jax import lax
from jax.experimental import pallas as pl
from jax.experimental.pallas import tpu as pltpu
from jax.experimental.pallas import tpu_sc as plsc

N = 10000
E = 320000
F = 128
C = 32
G = 128

TILES = 32
K = 128
PER = E // TILES
CH = 80
PERP = CH * K
SCH = CH // 2
NPAD = 10240
RPT = NPAD // 16
LANES = 16


def _zero_acc(buf, acc_sh, base):
    def zrow(i, _):
        for j in range(8):
            buf[i, pl.ds(j * LANES, LANES)] = jnp.zeros((LANES,), jnp.float32)
        return 0
    lax.fori_loop(0, K, zrow, 0)
    nfull, rem = RPT // K, RPT % K
    for i in range(nfull):
        pltpu.sync_copy(buf, acc_sh.at[pl.ds(base + i * K, K)])
    if rem:
        pltpu.sync_copy(buf.at[pl.ds(0, rem)],
                        acc_sh.at[pl.ds(base + nfull * K, rem)])


def _sc_agg_body(xwd_hbm, rows_hbm, ews_hbm, cols_hbm, out_hbm,
                 cidx_all, ri_a, ri_b, ew_a, ew_b, buf_a, buf_b, acc_sh,
                 sem_ga, sem_gb, sem_sa, sem_sb, sem_ia, sem_ib):
    c = lax.axis_index("c")
    s = lax.axis_index("s")
    base = s * RPT
    w = c * 16 + s

    def fire_idx(ci, ri, ew, sem):
        pltpu.async_copy(rows_hbm.at[w, ci], ri, sem)
        pltpu.async_copy(ews_hbm.at[w, ci], ew, sem)

    def drain_idx(ci, ri, ew, sem):
        pltpu.make_async_copy(rows_hbm.at[w, ci], ri, sem).wait()
        pltpu.make_async_copy(ews_hbm.at[w, ci], ew, sem).wait()

    def fire_gather(ri, buf, sem):
        pltpu.async_copy(xwd_hbm.at[ri], buf, sem)

    def drain_gather(ri, buf, sem):
        pltpu.make_async_copy(xwd_hbm.at[ri], buf, sem).wait()

    def fire_scatter(ci, buf, sem):
        pltpu.async_copy(buf, acc_sh.at[cidx_all.at[ci]], sem, add=True)

    def drain_scatter(ci, buf, sem):
        pltpu.make_async_copy(buf, acc_sh.at[cidx_all.at[ci]], sem).wait()

    def scale(buf, ew):
        def grp(g, _):
            off = g * LANES
            ev = ew[pl.ds(off, LANES)]
            for j in range(LANES):
                e = ev[j]
                for jc in range(8):
                    sl = pl.ds(jc * LANES, LANES)
                    buf[off + j, sl] = buf[off + j, sl] * e
            return 0
        lax.fori_loop(0, K // LANES, grp, 0)

    fire_idx(0, ri_a, ew_a, sem_ia)
    fire_idx(1, ri_b, ew_b, sem_ib)
    drain_idx(0, ri_a, ew_a, sem_ia)
    fire_gather(ri_a, buf_a, sem_ga)
    _zero_acc(buf_b, acc_sh, base)
    pltpu.sync_copy(cols_hbm.at[w], cidx_all)
    plsc.subcore_barrier()

    def body(i, _):
        c0 = 2 * i
        c1 = 2 * i + 1
        drain_gather(ri_a, buf_a, sem_ga)
        scale(buf_a, ew_a)

        @pl.when(i > 0)
        def _():
            drain_scatter(c1, buf_b, sem_sb)
        fire_scatter(c0, buf_a, sem_sa)
        drain_idx(c1, ri_b, ew_b, sem_ib)
        fire_gather(ri_b, buf_b, sem_gb)

        @pl.when(i < SCH - 1)
        def _():
            fire_idx(c0 + 2, ri_a, ew_a, sem_ia)
        drain_gather(ri_b, buf_b, sem_gb)
        scale(buf_b, ew_b)
        drain_scatter(c0, buf_a, sem_sa)
        fire_scatter(c1, buf_b, sem_sb)

        @pl.when(i < SCH - 1)
        def _():
            drain_idx(c0 + 2, ri_a, ew_a, sem_ia)
            fire_gather(ri_a, buf_a, sem_ga)
            fire_idx(c1 + 2, ri_b, ew_b, sem_ib)
        return 0
    lax.fori_loop(0, SCH, body, 0)
    drain_scatter(CH - 1, buf_b, sem_sb)

    plsc.subcore_barrier()
    pltpu.sync_copy(acc_sh.at[pl.ds(base, RPT)],
                    out_hbm.at[c, pl.ds(base, RPT)])


def _sc_agg(xwd, rows3, ews3, cols3):
    mesh = plsc.VectorSubcoreMesh(core_axis_name="c", subcore_axis_name="s")
    fn = functools.partial(
        pl.kernel, mesh=mesh,
        out_type=jax.ShapeDtypeStruct((2, NPAD, F), jnp.float32),
        scratch_types=[
            pltpu.VMEM((CH, K), jnp.int32),
            pltpu.VMEM((K,), jnp.int32),
            pltpu.VMEM((K,), jnp.int32),
            pltpu.VMEM((K,), jnp.float32),
            pltpu.VMEM((K,), jnp.float32),
            pltpu.VMEM((K, F), jnp.float32),
            pltpu.VMEM((K, F), jnp.float32),
            pltpu.VMEM_SHARED((NPAD, F), jnp.float32),
            pltpu.SemaphoreType.DMA,
            pltpu.SemaphoreType.DMA,
            pltpu.SemaphoreType.DMA,
            pltpu.SemaphoreType.DMA,
            pltpu.SemaphoreType.DMA,
            pltpu.SemaphoreType.DMA,
        ],
    )(_sc_agg_body)
    return fn(xwd, rows3, ews3, cols3)


FD = F


def _sc_deg_body(ews_hbm, cols_hbm, out_hbm,
                 cidx_all, ew_a, ew_b, buf_a, buf_b, acc_sh,
                 sem_sa, sem_sb, sem_ia, sem_ib):
    c = lax.axis_index("c")
    s = lax.axis_index("s")
    base = s * RPT
    w = c * 16 + s

    _zero_acc(buf_a, acc_sh, base)
    pltpu.sync_copy(cols_hbm.at[w], cidx_all)
    plsc.subcore_barrier()

    def fire_idx(ci, ew, sem):
        pltpu.async_copy(ews_hbm.at[w, ci], ew, sem)

    def drain_idx(ci, ew, sem):
        pltpu.make_async_copy(ews_hbm.at[w, ci], ew, sem).wait()

    def fire_scatter(ci, buf, sem):
        pltpu.async_copy(buf, acc_sh.at[cidx_all.at[ci]], sem, add=True)

    def drain_scatter(ci, buf, sem):
        pltpu.make_async_copy(buf, acc_sh.at[cidx_all.at[ci]], sem).wait()

    def build(buf, ew):
        def grp(g, _):
            off = g * LANES
            ev = ew[pl.ds(off, LANES)]
            for j in range(LANES):
                e16 = jnp.full((LANES,), ev[j], jnp.float32)
                for jc in range(FD // LANES):
                    buf[off + j, pl.ds(jc * LANES, LANES)] = e16
            return 0
        lax.fori_loop(0, K // LANES, grp, 0)

    fire_idx(0, ew_a, sem_ia)
    fire_idx(1, ew_b, sem_ib)

    def body(i, _):
        c0 = 2 * i
        c1 = 2 * i + 1
        drain_idx(c0, ew_a, sem_ia)
        build(buf_a, ew_a)

        @pl.when(i > 0)
        def _():
            drain_scatter(c1, buf_b, sem_sb)
        fire_scatter(c0, buf_a, sem_sa)
        drain_idx(c1, ew_b, sem_ib)
        build(buf_b, ew_b)
        drain_scatter(c0, buf_a, sem_sa)
        fire_scatter(c1, buf_b, sem_sb)

        @pl.when(i < SCH - 1)
        def _():
            fire_idx(c0 + 2, ew_a, sem_ia)
            fire_idx(c1 + 2, ew_b, sem_ib)
        return 0
    lax.fori_loop(0, SCH, body, 0)
    drain_scatter(CH - 1, buf_b, sem_sb)

    plsc.subcore_barrier()
    pltpu.sync_copy(acc_sh.at[pl.ds(base, RPT)],
                    out_hbm.at[c, pl.ds(base, RPT)])


def _sc_deg(ews3, cols3):
    mesh = plsc.VectorSubcoreMesh(core_axis_name="c", subcore_axis_name="s")
    fn = functools.partial(
        pl.kernel, mesh=mesh,
        out_type=jax.ShapeDtypeStruct((2, NPAD, FD), jnp.float32),
        scratch_types=[
            pltpu.VMEM((CH, K), jnp.int32),
            pltpu.VMEM((K,), jnp.float32),
            pltpu.VMEM((K,), jnp.float32),
            pltpu.VMEM((K, FD), jnp.float32),
            pltpu.VMEM((K, FD), jnp.float32),
            pltpu.VMEM_SHARED((NPAD, FD), jnp.float32),
            pltpu.SemaphoreType.DMA,
            pltpu.SemaphoreType.DMA,
            pltpu.SemaphoreType.DMA,
            pltpu.SemaphoreType.DMA,
        ],
    )(_sc_deg_body)
    return fn(ews3, cols3)


def _sumsq_body(x_ref, o_ref):
    blk = x_ref[...]
    o_ref[...] = jnp.sum(blk * blk).reshape(1, 1)


def _sumsq(ew2d):
    n = ew2d.shape[0]
    return pl.pallas_call(
        _sumsq_body,
        out_shape=jax.ShapeDtypeStruct((1, 1), jnp.float32),
        grid=(1,),
        in_specs=[pl.BlockSpec((n, 128), lambda i: (0, 0))],
        out_specs=pl.BlockSpec((1, 1), lambda i: (0, 0)),
    )(ew2d)


def _prep_body(degp_ref, s2_ref, o_ref):
    s = lax.rsqrt(jnp.maximum(s2_ref[...][0, 0], 1e-24))
    d = degp_ref[...]
    deg = s * (d[0] + d[1]) + 1.0
    o_ref[...] = lax.rsqrt(deg)


def _prep(degp, s2):
    blk = 2000
    return pl.pallas_call(
        _prep_body,
        out_shape=jax.ShapeDtypeStruct((N, F), jnp.float32),
        grid=(N // blk,),
        in_specs=[
            pl.BlockSpec((2, blk, FD), lambda i: (0, i, 0)),
            pl.BlockSpec((1, 1), lambda i: (0, 0)),
        ],
        out_specs=pl.BlockSpec((blk, F), lambda i: (i, 0)),
    )(degp, s2)


def _mm1_body(x_ref, w_ref, dis_ref, o_ref):
    xw = jax.lax.dot_general(x_ref[...], w_ref[...], (((1,), (0,)), ((), ())),
                             precision=lax.Precision.HIGHEST,
                             preferred_element_type=jnp.float32)
    o_ref[...] = dis_ref[...] * xw


def _mm1(x, W, dis):
    blk = 2000
    fin = x.shape[1]
    return pl.pallas_call(
        _mm1_body,
        out_shape=jax.ShapeDtypeStruct((N, F), jnp.float32),
        grid=(N // blk,),
        in_specs=[
            pl.BlockSpec((blk, fin), lambda i: (i, 0)),
            pl.BlockSpec((fin, F), lambda i: (0, 0)),
            pl.BlockSpec((blk, F), lambda i: (i, 0)),
        ],
        out_specs=pl.BlockSpec((blk, F), lambda i: (i, 0)),
    )(x, W, dis)


def _layer_body(agg_ref, xwd_ref, dis_ref, s2_ref, b_ref, wn_ref, o_ref):
    s = lax.rsqrt(jnp.maximum(s2_ref[...][0, 0], 1e-24))
    dis = dis_ref[...]
    h = dis * (s * (agg_ref[0] + agg_ref[1]) + xwd_ref[...]) + b_ref[...]
    h = jnp.maximum(h, 0.0)
    hw = jax.lax.dot_general(h, wn_ref[...], (((1,), (0,)), ((), ())),
                             precision=lax.Precision.HIGHEST,
                             preferred_element_type=jnp.float32)
    o_ref[...] = dis * hw


def _layer(agg, xwd, dis, s2, b, Wn):
    blk = 2000
    return pl.pallas_call(
        _layer_body,
        out_shape=jax.ShapeDtypeStruct((N, F), jnp.float32),
        grid=(N // blk,),
        in_specs=[
            pl.BlockSpec((2, blk, F), lambda i: (0, i, 0)),
            pl.BlockSpec((blk, F), lambda i: (i, 0)),
            pl.BlockSpec((blk, F), lambda i: (i, 0)),
            pl.BlockSpec((1, 1), lambda i: (0, 0)),
            pl.BlockSpec((1, F), lambda i: (0, 0)),
            pl.BlockSpec((F, F), lambda i: (0, 0)),
        ],
        out_specs=pl.BlockSpec((blk, F), lambda i: (i, 0)),
    )(agg, xwd, dis, s2, b, Wn)


def _pool_body(agg_ref, xwd_ref, dis_ref, s2_ref, b_ref, batch_ref, o_ref):
    @pl.when(pl.program_id(0) == 0)
    def _():
        o_ref[...] = jnp.zeros_like(o_ref)
    s = lax.rsqrt(jnp.maximum(s2_ref[...][0, 0], 1e-24))
    dis = dis_ref[...]
    h = dis * (s * (agg_ref[0] + agg_ref[1]) + xwd_ref[...]) + b_ref[...]
    h = jnp.maximum(h, 0.0)
    gids = jax.lax.broadcasted_iota(jnp.int32, (1, G), 1)
    onehot = (batch_ref[...] == gids).astype(jnp.float32)
    o_ref[...] += jax.lax.dot_general(
        onehot, h, (((0,), (0,)), ((), ())),
        precision=lax.Precision.HIGHEST,
        preferred_element_type=jnp.float32)


def _pool(agg, xwd, dis, s2, b, batch2d):
    blk = 2000
    return pl.pallas_call(
        _pool_body,
        out_shape=jax.ShapeDtypeStruct((G, F), jnp.float32),
        grid=(N // blk,),
        in_specs=[
            pl.BlockSpec((2, blk, F), lambda i: (0, i, 0)),
            pl.BlockSpec((blk, F), lambda i: (i, 0)),
            pl.BlockSpec((blk, F), lambda i: (i, 0)),
            pl.BlockSpec((1, 1), lambda i: (0, 0)),
            pl.BlockSpec((1, F), lambda i: (0, 0)),
            pl.BlockSpec((blk, 1), lambda i: (i, 0)),
        ],
        out_specs=pl.BlockSpec((G, F), lambda i: (0, 0)),
    )(agg, xwd, dis, s2, b, batch2d)


def _head_body(p_ref, w1_ref, b1_ref, w2_ref, b2_ref, o_ref):
    h1 = jax.lax.dot_general(p_ref[...], w1_ref[...], (((1,), (0,)), ((), ())),
                             precision=lax.Precision.HIGHEST,
                             preferred_element_type=jnp.float32)
    h1 = jnp.maximum(h1 + b1_ref[...], 0.0)
    t = jax.lax.dot_general(h1, w2_ref[...], (((1,), (0,)), ((), ())),
                            precision=lax.Precision.HIGHEST,
                            preferred_element_type=jnp.float32) + b2_ref[...]
    m = jnp.max(t, axis=-1, keepdims=True)
    lse = jnp.log(jnp.sum(jnp.exp(t - m), axis=-1, keepdims=True)) + m
    o_ref[...] = t - lse


def _head(pooled, Wl1, bl1, Wl2p, bl2p):
    return pl.pallas_call(
        _head_body,
        out_shape=jax.ShapeDtypeStruct((G, F), jnp.float32),
        in_specs=[pl.BlockSpec(a.shape, lambda: tuple(0 for _ in a.shape))
                  for a in (pooled, Wl1, bl1, Wl2p, bl2p)],
        out_specs=pl.BlockSpec((G, F), lambda: (0, 0)),
    )(pooled, Wl1, bl1, Wl2p, bl2p)


def kernel(x, edge_index, edge_weight, batch,
           W1, b1, W2, b2, W3, b3, W4, b4, W5, b5, Wl1, bl1, Wl2, bl2):
    pad = TILES * PERP - E
    spread = jnp.arange(pad, dtype=jnp.int32)
    row = jnp.concatenate([edge_index[0], spread % N])
    col = jnp.concatenate([edge_index[1], spread % NPAD])
    ewp = jnp.concatenate([edge_weight, jnp.zeros((pad,), jnp.float32)])
    rows3 = row.reshape(TILES, CH, K)
    cols3 = col.reshape(TILES, CH, K)
    ews3 = ewp.reshape(TILES, CH, K)

    s2 = _sumsq(edge_weight.reshape(2500, 128))
    degp = _sc_deg(ews3, cols3)
    dis = _prep(degp, s2)

    batch2d = batch.reshape(N, 1)
    bs = [b1, b2, b3, b4, b5]
    Ws = [W2, W3, W4, W5]

    xwd = _mm1(x, W1, dis)
    for l in range(4):
        agg = _sc_agg(xwd, rows3, ews3, cols3)
        xwd = _layer(agg, xwd, dis, s2, bs[l].reshape(1, F), Ws[l])
    agg = _sc_agg(xwd, rows3, ews3, cols3)
    pooled = _pool(agg, xwd, dis, s2, bs[4].reshape(1, F), batch2d)

    Wl2p = jnp.zeros((F, F), jnp.float32).at[:, :C].set(Wl2)
    bl2p = jnp.full((1, F), -1e30, jnp.float32).at[0, :C].set(bl2)
    out = _head(pooled, Wl1, bl1.reshape(1, F), Wl2p, bl2p)
    return out[:, :C]


# speedup vs baseline: 1.2751x; 1.2399x over previous
import functools

import jax
import jax.numpy as jnp
from jax import lax
from jax.experimental import pallas as pl
from jax.experimental.pallas import tpu as pltpu
from jax.experimental.pallas import tpu_sc as plsc

N = 10000
E = 320000
F = 128
C = 32
G = 128

TILES = 32
K = 128
PER = E // TILES
CH = 80
PERP = CH * K
SCH = CH // 2
NPAD = 10240
RPT = NPAD // 16
LANES = 16


def _zero_acc(buf, acc_sh, base):
    def zrow(i, _):
        for j in range(8):
            buf[i, pl.ds(j * LANES, LANES)] = jnp.zeros((LANES,), jnp.float32)
        return 0
    lax.fori_loop(0, K, zrow, 0)
    nfull, rem = RPT // K, RPT % K
    for i in range(nfull):
        pltpu.sync_copy(buf, acc_sh.at[pl.ds(base + i * K, K)])
    if rem:
        pltpu.sync_copy(buf.at[pl.ds(0, rem)],
                        acc_sh.at[pl.ds(base + nfull * K, rem)])


def _sc_agg_body(xwd_hbm, rows_hbm, ews_hbm, cols_hbm, out_hbm,
                 cidx_all, ri_a, ri_b, ew_a, ew_b, buf_a, buf_b, acc_sh,
                 sem_ga, sem_gb, sem_sa, sem_sb, sem_ia, sem_ib):
    c = lax.axis_index("c")
    s = lax.axis_index("s")
    base = s * RPT
    w = c * 16 + s

    def fire_idx(ci, ri, ew, sem):
        pltpu.async_copy(rows_hbm.at[w, ci], ri, sem)
        pltpu.async_copy(ews_hbm.at[w, ci], ew, sem)

    def drain_idx(ci, ri, ew, sem):
        pltpu.make_async_copy(rows_hbm.at[w, ci], ri, sem).wait()
        pltpu.make_async_copy(ews_hbm.at[w, ci], ew, sem).wait()

    def fire_gather(ri, buf, sem):
        pltpu.async_copy(xwd_hbm.at[ri], buf, sem)

    def drain_gather(ri, buf, sem):
        pltpu.make_async_copy(xwd_hbm.at[ri], buf, sem).wait()

    def fire_scatter(ci, buf, sem):
        pltpu.async_copy(buf, acc_sh.at[cidx_all.at[ci]], sem, add=True)

    def drain_scatter(ci, buf, sem):
        pltpu.make_async_copy(buf, acc_sh.at[cidx_all.at[ci]], sem).wait()

    def scale(buf, ew):
        def grp(g, _):
            off = g * LANES
            ev = ew[pl.ds(off, LANES)]
            for j in range(LANES):
                e = ev[j]
                for jc in range(8):
                    sl = pl.ds(jc * LANES, LANES)
                    buf[off + j, sl] = buf[off + j, sl] * e
            return 0
        lax.fori_loop(0, K // LANES, grp, 0)

    fire_idx(0, ri_a, ew_a, sem_ia)
    fire_idx(1, ri_b, ew_b, sem_ib)
    drain_idx(0, ri_a, ew_a, sem_ia)
    fire_gather(ri_a, buf_a, sem_ga)
    _zero_acc(buf_b, acc_sh, base)
    pltpu.sync_copy(cols_hbm.at[w], cidx_all)
    plsc.subcore_barrier()

    def body(i, _):
        c0 = 2 * i
        c1 = 2 * i + 1
        drain_gather(ri_a, buf_a, sem_ga)

        @pl.when(i > 0)
        def _():
            drain_scatter(c1, buf_b, sem_sb)
        drain_idx(c1, ri_b, ew_b, sem_ib)
        fire_gather(ri_b, buf_b, sem_gb)
        scale(buf_a, ew_a)
        fire_scatter(c0, buf_a, sem_sa)

        @pl.when(i < SCH - 1)
        def _():
            fire_idx(c0 + 2, ri_a, ew_a, sem_ia)
        drain_gather(ri_b, buf_b, sem_gb)
        drain_scatter(c0, buf_a, sem_sa)

        @pl.when(i < SCH - 1)
        def _():
            drain_idx(c0 + 2, ri_a, ew_a, sem_ia)
            fire_gather(ri_a, buf_a, sem_ga)
        scale(buf_b, ew_b)
        fire_scatter(c1, buf_b, sem_sb)

        @pl.when(i < SCH - 1)
        def _():
            fire_idx(c1 + 2, ri_b, ew_b, sem_ib)
        return 0
    lax.fori_loop(0, SCH, body, 0)
    drain_scatter(CH - 1, buf_b, sem_sb)

    plsc.subcore_barrier()
    pltpu.sync_copy(acc_sh.at[pl.ds(base, RPT)],
                    out_hbm.at[c, pl.ds(base, RPT)])


def _sc_agg(xwd, rows3, ews3, cols3):
    mesh = plsc.VectorSubcoreMesh(core_axis_name="c", subcore_axis_name="s")
    fn = functools.partial(
        pl.kernel, mesh=mesh,
        out_type=jax.ShapeDtypeStruct((2, NPAD, F), jnp.float32),
        scratch_types=[
            pltpu.VMEM((CH, K), jnp.int32),
            pltpu.VMEM((K,), jnp.int32),
            pltpu.VMEM((K,), jnp.int32),
            pltpu.VMEM((K,), jnp.float32),
            pltpu.VMEM((K,), jnp.float32),
            pltpu.VMEM((K, F), jnp.float32),
            pltpu.VMEM((K, F), jnp.float32),
            pltpu.VMEM_SHARED((NPAD, F), jnp.float32),
            pltpu.SemaphoreType.DMA,
            pltpu.SemaphoreType.DMA,
            pltpu.SemaphoreType.DMA,
            pltpu.SemaphoreType.DMA,
            pltpu.SemaphoreType.DMA,
            pltpu.SemaphoreType.DMA,
        ],
    )(_sc_agg_body)
    return fn(xwd, rows3, ews3, cols3)


FD = F


def _sc_deg_body(ews_hbm, cols_hbm, out_hbm,
                 cidx_all, ew_a, ew_b, buf_a, buf_b, acc_sh,
                 sem_sa, sem_sb, sem_ia, sem_ib):
    c = lax.axis_index("c")
    s = lax.axis_index("s")
    base = s * RPT
    w = c * 16 + s

    _zero_acc(buf_a, acc_sh, base)
    pltpu.sync_copy(cols_hbm.at[w], cidx_all)
    plsc.subcore_barrier()

    def fire_idx(ci, ew, sem):
        pltpu.async_copy(ews_hbm.at[w, ci], ew, sem)

    def drain_idx(ci, ew, sem):
        pltpu.make_async_copy(ews_hbm.at[w, ci], ew, sem).wait()

    def fire_scatter(ci, buf, sem):
        pltpu.async_copy(buf, acc_sh.at[cidx_all.at[ci]], sem, add=True)

    def drain_scatter(ci, buf, sem):
        pltpu.make_async_copy(buf, acc_sh.at[cidx_all.at[ci]], sem).wait()

    def build(buf, ew):
        def grp(g, _):
            off = g * LANES
            ev = ew[pl.ds(off, LANES)]
            for j in range(LANES):
                e16 = jnp.full((LANES,), ev[j], jnp.float32)
                for jc in range(FD // LANES):
                    buf[off + j, pl.ds(jc * LANES, LANES)] = e16
            return 0
        lax.fori_loop(0, K // LANES, grp, 0)

    fire_idx(0, ew_a, sem_ia)
    fire_idx(1, ew_b, sem_ib)

    def body(i, _):
        c0 = 2 * i
        c1 = 2 * i + 1
        drain_idx(c0, ew_a, sem_ia)
        build(buf_a, ew_a)

        @pl.when(i > 0)
        def _():
            drain_scatter(c1, buf_b, sem_sb)
        fire_scatter(c0, buf_a, sem_sa)
        drain_idx(c1, ew_b, sem_ib)
        build(buf_b, ew_b)
        drain_scatter(c0, buf_a, sem_sa)
        fire_scatter(c1, buf_b, sem_sb)

        @pl.when(i < SCH - 1)
        def _():
            fire_idx(c0 + 2, ew_a, sem_ia)
            fire_idx(c1 + 2, ew_b, sem_ib)
        return 0
    lax.fori_loop(0, SCH, body, 0)
    drain_scatter(CH - 1, buf_b, sem_sb)

    plsc.subcore_barrier()
    pltpu.sync_copy(acc_sh.at[pl.ds(base, RPT)],
                    out_hbm.at[c, pl.ds(base, RPT)])


def _sc_deg(ews3, cols3):
    mesh = plsc.VectorSubcoreMesh(core_axis_name="c", subcore_axis_name="s")
    fn = functools.partial(
        pl.kernel, mesh=mesh,
        out_type=jax.ShapeDtypeStruct((2, NPAD, FD), jnp.float32),
        scratch_types=[
            pltpu.VMEM((CH, K), jnp.int32),
            pltpu.VMEM((K,), jnp.float32),
            pltpu.VMEM((K,), jnp.float32),
            pltpu.VMEM((K, FD), jnp.float32),
            pltpu.VMEM((K, FD), jnp.float32),
            pltpu.VMEM_SHARED((NPAD, FD), jnp.float32),
            pltpu.SemaphoreType.DMA,
            pltpu.SemaphoreType.DMA,
            pltpu.SemaphoreType.DMA,
            pltpu.SemaphoreType.DMA,
        ],
    )(_sc_deg_body)
    return fn(ews3, cols3)


def _sumsq_body(x_ref, o_ref):
    blk = x_ref[...]
    o_ref[...] = jnp.sum(blk * blk).reshape(1, 1)


def _sumsq(ew2d):
    n = ew2d.shape[0]
    return pl.pallas_call(
        _sumsq_body,
        out_shape=jax.ShapeDtypeStruct((1, 1), jnp.float32),
        grid=(1,),
        in_specs=[pl.BlockSpec((n, 128), lambda i: (0, 0))],
        out_specs=pl.BlockSpec((1, 1), lambda i: (0, 0)),
    )(ew2d)


def _prep_body(degp_ref, s2_ref, o_ref):
    s = lax.rsqrt(jnp.maximum(s2_ref[...][0, 0], 1e-24))
    d = degp_ref[...]
    deg = s * (d[0] + d[1]) + 1.0
    o_ref[...] = lax.rsqrt(deg)


def _prep(degp, s2):
    blk = 2000
    return pl.pallas_call(
        _prep_body,
        out_shape=jax.ShapeDtypeStruct((N, F), jnp.float32),
        grid=(N // blk,),
        in_specs=[
            pl.BlockSpec((2, blk, FD), lambda i: (0, i, 0)),
            pl.BlockSpec((1, 1), lambda i: (0, 0)),
        ],
        out_specs=pl.BlockSpec((blk, F), lambda i: (i, 0)),
    )(degp, s2)


def _mm1_body(x_ref, w_ref, dis_ref, o_ref):
    xw = jax.lax.dot_general(x_ref[...], w_ref[...], (((1,), (0,)), ((), ())),
                             precision=lax.Precision.HIGHEST,
                             preferred_element_type=jnp.float32)
    o_ref[...] = dis_ref[...] * xw


def _mm1(x, W, dis):
    blk = 2000
    fin = x.shape[1]
    return pl.pallas_call(
        _mm1_body,
        out_shape=jax.ShapeDtypeStruct((N, F), jnp.float32),
        grid=(N // blk,),
        in_specs=[
            pl.BlockSpec((blk, fin), lambda i: (i, 0)),
            pl.BlockSpec((fin, F), lambda i: (0, 0)),
            pl.BlockSpec((blk, F), lambda i: (i, 0)),
        ],
        out_specs=pl.BlockSpec((blk, F), lambda i: (i, 0)),
    )(x, W, dis)


def _layer_body(agg_ref, xwd_ref, dis_ref, s2_ref, b_ref, wn_ref, o_ref):
    s = lax.rsqrt(jnp.maximum(s2_ref[...][0, 0], 1e-24))
    dis = dis_ref[...]
    h = dis * (s * (agg_ref[0] + agg_ref[1]) + xwd_ref[...]) + b_ref[...]
    h = jnp.maximum(h, 0.0)
    hw = jax.lax.dot_general(h, wn_ref[...], (((1,), (0,)), ((), ())),
                             precision=lax.Precision.HIGHEST,
                             preferred_element_type=jnp.float32)
    o_ref[...] = dis * hw


def _layer(agg, xwd, dis, s2, b, Wn):
    blk = 2000
    return pl.pallas_call(
        _layer_body,
        out_shape=jax.ShapeDtypeStruct((N, F), jnp.float32),
        grid=(N // blk,),
        in_specs=[
            pl.BlockSpec((2, blk, F), lambda i: (0, i, 0)),
            pl.BlockSpec((blk, F), lambda i: (i, 0)),
            pl.BlockSpec((blk, F), lambda i: (i, 0)),
            pl.BlockSpec((1, 1), lambda i: (0, 0)),
            pl.BlockSpec((1, F), lambda i: (0, 0)),
            pl.BlockSpec((F, F), lambda i: (0, 0)),
        ],
        out_specs=pl.BlockSpec((blk, F), lambda i: (i, 0)),
    )(agg, xwd, dis, s2, b, Wn)


def _pool_body(agg_ref, xwd_ref, dis_ref, s2_ref, b_ref, batch_ref, o_ref):
    @pl.when(pl.program_id(0) == 0)
    def _():
        o_ref[...] = jnp.zeros_like(o_ref)
    s = lax.rsqrt(jnp.maximum(s2_ref[...][0, 0], 1e-24))
    dis = dis_ref[...]
    h = dis * (s * (agg_ref[0] + agg_ref[1]) + xwd_ref[...]) + b_ref[...]
    h = jnp.maximum(h, 0.0)
    gids = jax.lax.broadcasted_iota(jnp.int32, (1, G), 1)
    onehot = (batch_ref[...] == gids).astype(jnp.float32)
    o_ref[...] += jax.lax.dot_general(
        onehot, h, (((0,), (0,)), ((), ())),
        precision=lax.Precision.HIGHEST,
        preferred_element_type=jnp.float32)


def _pool(agg, xwd, dis, s2, b, batch2d):
    blk = 2000
    return pl.pallas_call(
        _pool_body,
        out_shape=jax.ShapeDtypeStruct((G, F), jnp.float32),
        grid=(N // blk,),
        in_specs=[
            pl.BlockSpec((2, blk, F), lambda i: (0, i, 0)),
            pl.BlockSpec((blk, F), lambda i: (i, 0)),
            pl.BlockSpec((blk, F), lambda i: (i, 0)),
            pl.BlockSpec((1, 1), lambda i: (0, 0)),
            pl.BlockSpec((1, F), lambda i: (0, 0)),
            pl.BlockSpec((blk, 1), lambda i: (i, 0)),
        ],
        out_specs=pl.BlockSpec((G, F), lambda i: (0, 0)),
    )(agg, xwd, dis, s2, b, batch2d)


def _head_body(p_ref, w1_ref, b1_ref, w2_ref, b2_ref, o_ref):
    h1 = jax.lax.dot_general(p_ref[...], w1_ref[...], (((1,), (0,)), ((), ())),
                             precision=lax.Precision.HIGHEST,
                             preferred_element_type=jnp.float32)
    h1 = jnp.maximum(h1 + b1_ref[...], 0.0)
    t = jax.lax.dot_general(h1, w2_ref[...], (((1,), (0,)), ((), ())),
                            precision=lax.Precision.HIGHEST,
                            preferred_element_type=jnp.float32) + b2_ref[...]
    m = jnp.max(t, axis=-1, keepdims=True)
    lse = jnp.log(jnp.sum(jnp.exp(t - m), axis=-1, keepdims=True)) + m
    o_ref[...] = t - lse


def _head(pooled, Wl1, bl1, Wl2p, bl2p):
    return pl.pallas_call(
        _head_body,
        out_shape=jax.ShapeDtypeStruct((G, F), jnp.float32),
        in_specs=[pl.BlockSpec(a.shape, lambda: tuple(0 for _ in a.shape))
                  for a in (pooled, Wl1, bl1, Wl2p, bl2p)],
        out_specs=pl.BlockSpec((G, F), lambda: (0, 0)),
    )(pooled, Wl1, bl1, Wl2p, bl2p)


def kernel(x, edge_index, edge_weight, batch,
           W1, b1, W2, b2, W3, b3, W4, b4, W5, b5, Wl1, bl1, Wl2, bl2):
    pad = TILES * PERP - E
    spread = jnp.arange(pad, dtype=jnp.int32)
    row = jnp.concatenate([edge_index[0], spread % N])
    col = jnp.concatenate([edge_index[1], spread % NPAD])
    ewp = jnp.concatenate([edge_weight, jnp.zeros((pad,), jnp.float32)])
    rows3 = row.reshape(TILES, CH, K)
    cols3 = col.reshape(TILES, CH, K)
    ews3 = ewp.reshape(TILES, CH, K)

    s2 = _sumsq(edge_weight.reshape(2500, 128))
    degp = _sc_deg(ews3, cols3)
    dis = _prep(degp, s2)

    batch2d = batch.reshape(N, 1)
    bs = [b1, b2, b3, b4, b5]
    Ws = [W2, W3, W4, W5]

    xwd = _mm1(x, W1, dis)
    for l in range(4):
        agg = _sc_agg(xwd, rows3, ews3, cols3)
        xwd = _layer(agg, xwd, dis, s2, bs[l].reshape(1, F), Ws[l])
    agg = _sc_agg(xwd, rows3, ews3, cols3)
    pooled = _pool(agg, xwd, dis, s2, bs[4].reshape(1, F), batch2d)

    Wl2p = jnp.zeros((F, F), jnp.float32).at[:, :C].set(Wl2)
    bl2p = jnp.full((1, F), -1e30, jnp.float32).at[0, :C].set(bl2)
    out = _head(pooled, Wl1, bl1.reshape(1, F), Wl2p, bl2p)
    return out[:, :C]
